# initial kernel scaffold (unmeasured)
import jax
import jax.numpy as jnp
from jax import lax
from jax.experimental import pallas as pl
from jax.experimental.pallas import tpu as pltpu


def kernel(
    x,
):
    def body(*refs):
        pass

    out_shape = jax.ShapeDtypeStruct(..., jnp.float32)
    return pl.pallas_call(body, out_shape=out_shape)(...)


# baseline (device time: 32232 ns/iter reference)
import jax
import jax.numpy as jnp
from jax import lax
from jax.experimental import pallas as pl
from jax.experimental.pallas import tpu as pltpu

M, N = 2048, 512
HALF = M // 2


def kernel(x):
    def body(x_ref, out_ref, send_buf, recv_buf, sems):
        my_x = lax.axis_index("x")
        my_y = lax.axis_index("y")

        barrier = pltpu.get_barrier_semaphore()
        pl.semaphore_signal(
            barrier, inc=1, device_id=(my_x, 1 - my_y),
            device_id_type=pl.DeviceIdType.MESH,
        )
        pl.semaphore_signal(
            barrier, inc=1, device_id=(1 - my_x, my_y),
            device_id_type=pl.DeviceIdType.MESH,
        )
        pl.semaphore_wait(barrier, 2)

        off = my_x * HALF
        send_buf[...] = x_ref[pl.ds(off, HALF), :].astype(jnp.bfloat16)

        rdma1 = pltpu.make_async_remote_copy(
            src_ref=send_buf,
            dst_ref=recv_buf,
            send_sem=sems.at[0],
            recv_sem=sems.at[1],
            device_id=(my_x, 1 - my_y),
            device_id_type=pl.DeviceIdType.MESH,
        )
        rdma1.start()
        rdma1.wait()

        out_ref[pl.ds(off, HALF), :] = send_buf[...] + recv_buf[...]

        rdma2 = pltpu.make_async_remote_copy(
            src_ref=out_ref.at[pl.ds(off, HALF)],
            dst_ref=out_ref.at[pl.ds(off, HALF)],
            send_sem=sems.at[2],
            recv_sem=sems.at[3],
            device_id=(1 - my_x, my_y),
            device_id_type=pl.DeviceIdType.MESH,
        )
        rdma2.start()
        rdma2.wait()

    return pl.pallas_call(
        body,
        out_shape=jax.ShapeDtypeStruct((M, N), jnp.bfloat16),
        in_specs=[pl.BlockSpec(memory_space=pltpu.VMEM)],
        out_specs=pl.BlockSpec(memory_space=pltpu.VMEM),
        scratch_shapes=[
            pltpu.VMEM((HALF, N), jnp.bfloat16),
            pltpu.VMEM((HALF, N), jnp.bfloat16),
            pltpu.SemaphoreType.DMA((4,)),
        ],
        compiler_params=pltpu.CompilerParams(collective_id=0),
    )(x)


# device time: 22459 ns/iter; 1.4351x vs baseline; 1.4351x over previous
import jax
import jax.numpy as jnp
from jax import lax
from jax.experimental import pallas as pl
from jax.experimental.pallas import tpu as pltpu

M, N = 2048, 512
HALF = M // 2
C = 8
CH = HALF // C
CHUNKS = (CH,) * C
OFFS = tuple(sum(CHUNKS[:h]) for h in range(C))


def kernel(x):
    def body(x_ref, out_ref, sb, rb, s1, r1, s2, r2):
        my_x = lax.axis_index("x")
        my_y = lax.axis_index("y")
        off = my_x * HALF

        barrier = pltpu.get_barrier_semaphore()
        pl.semaphore_signal(
            barrier, inc=1, device_id=(my_x, 1 - my_y),
            device_id_type=pl.DeviceIdType.MESH,
        )
        pl.semaphore_signal(
            barrier, inc=1, device_id=(1 - my_x, my_y),
            device_id_type=pl.DeviceIdType.MESH,
        )
        pl.semaphore_wait(barrier, 2)

        p1 = []
        for h in range(C):
            sb[pl.ds(OFFS[h], CHUNKS[h]), :] = x_ref[
                pl.ds(off + OFFS[h], CHUNKS[h]), :
            ].astype(jnp.bfloat16)
            rdma = pltpu.make_async_remote_copy(
                src_ref=sb.at[pl.ds(OFFS[h], CHUNKS[h])],
                dst_ref=rb.at[pl.ds(OFFS[h], CHUNKS[h])],
                send_sem=s1.at[h],
                recv_sem=r1.at[h],
                device_id=(my_x, 1 - my_y),
                device_id_type=pl.DeviceIdType.MESH,
            )
            rdma.start()
            p1.append(rdma)

        p2 = []
        for h in range(C):
            p1[h].wait()
            out_ref[pl.ds(off + OFFS[h], CHUNKS[h]), :] = (
                sb[pl.ds(OFFS[h], CHUNKS[h]), :]
                + rb[pl.ds(OFFS[h], CHUNKS[h]), :]
            )
            rdma = pltpu.make_async_remote_copy(
                src_ref=out_ref.at[pl.ds(off + OFFS[h], CHUNKS[h])],
                dst_ref=out_ref.at[pl.ds(off + OFFS[h], CHUNKS[h])],
                send_sem=s2.at[h],
                recv_sem=r2.at[h],
                device_id=(1 - my_x, my_y),
                device_id_type=pl.DeviceIdType.MESH,
            )
            rdma.start()
            p2.append(rdma)

        for r in p2:
            r.wait()

    return pl.pallas_call(
        body,
        out_shape=jax.ShapeDtypeStruct((M, N), jnp.bfloat16),
        in_specs=[pl.BlockSpec(memory_space=pltpu.VMEM)],
        out_specs=pl.BlockSpec(memory_space=pltpu.VMEM),
        scratch_shapes=[
            pltpu.VMEM((HALF, N), jnp.bfloat16),
            pltpu.VMEM((HALF, N), jnp.bfloat16),
            pltpu.SemaphoreType.DMA((C,)),
            pltpu.SemaphoreType.DMA((C,)),
            pltpu.SemaphoreType.DMA((C,)),
            pltpu.SemaphoreType.DMA((C,)),
        ],
        compiler_params=pltpu.CompilerParams(collective_id=0),
    )(x)
